# baseline (speedup 1.0000x reference)
import functools

import jax
import jax.numpy as jnp
from jax import lax
from jax.experimental import pallas as pl
from jax.experimental.pallas import tpu as pltpu
from jax.experimental.pallas import tpu_sc as plsc

NC, NS, LANES = 2, 16, 16
NT = NC * NS

N, C, K, OUT = 10000, 128, 32, 128
NPAD = 10240
CPC = OUT // NT
BLKN = 128
NBLK = NPAD // BLKN
GROUPS = BLKN // LANES
BN_BLK = 1024
CBLK = 8


def _tables_body(xp_ref, we_ref, wo_ref, i1_ref, i0_ref,
                 ut_ref, vt_ref, ix_ref):
    we = we_ref[...]
    wo = wo_ref[...]
    xb = xp_ref[...]
    dn = (((1,), (0,)), ((), ()))

    def pack(mat_e, mat_o):
        e = lax.dot_general(mat_e, xb, dn, preferred_element_type=jnp.float32)
        o = lax.dot_general(mat_o, xb, dn, preferred_element_type=jnp.float32)
        el = lax.bitcast_convert_type(e.astype(jnp.bfloat16),
                                      jnp.uint16).astype(jnp.uint32)
        oh = lax.bitcast_convert_type(o.astype(jnp.bfloat16),
                                      jnp.uint16).astype(jnp.uint32)
        return (el | (oh << 16)).astype(jnp.int32)

    ut_ref[...] = pack(we[:, :C] - we[:, C:], wo[:, :C] - wo[:, C:])
    vt_ref[...] = pack(we[:, C:], wo[:, C:])

    def kmajor(a):
        return a.T.reshape(K, BN_BLK // BLKN, BLKN).transpose(1, 0, 2)

    ix_ref[...] = jnp.stack([kmajor(i1_ref[...]), kmajor(i0_ref[...])],
                            axis=1)


def _sc_body(idx_hbm, ut_hbm, vt_hbm, m_hbm, ps_hbm, pq_hbm,
             utile, vtile, idx0, idx1, mb0, mb1, statbuf,
             sem_i0, sem_i1, sem_w):
    cid = lax.axis_index("c")
    sid = lax.axis_index("s")
    wid = sid * NC + cid

    pltpu.sync_copy(ut_hbm.at[2 * wid], utile.at[pl.ds(0, NPAD)])
    pltpu.sync_copy(ut_hbm.at[2 * wid + 1], utile.at[pl.ds(NPAD, NPAD)])
    pltpu.sync_copy(vt_hbm.at[2 * wid], vtile.at[pl.ds(0, NPAD)])
    pltpu.sync_copy(vt_hbm.at[2 * wid + 1], vtile.at[pl.ds(NPAD, NPAD)])

    ibufs = ((idx0, sem_i0), (idx1, sem_i1))
    mbufs = (mb0, mb1)

    def idx_start(blk, slot):
        buf, sem = ibufs[slot]
        pltpu.make_async_copy(idx_hbm.at[blk], buf, sem).start()

    def idx_drain(blk, slot):
        buf, sem = ibufs[slot]
        pltpu.make_async_copy(idx_hbm.at[blk], buf, sem).wait()

    zero = jnp.zeros((LANES,), jnp.float32)
    neg = jnp.full((2 * LANES,), -jnp.inf, jnp.bfloat16)
    cnp = jnp.full((LANES,), NPAD, jnp.int32)
    himask = jnp.int32(-65536)
    bzero = jnp.zeros((2 * LANES,), jnp.bfloat16)

    idx_start(0, 0)

    def blk_pair(bg, carry):
        acc = carry
        for slot in range(2):
            blk = bg * 2 + slot
            bn = jnp.minimum(blk + 1, NBLK - 1)
            idx_start(bn, 1 - slot)
            idx_drain(blk, slot)
            idxb, _ = ibufs[slot]
            mblk = mbufs[slot]
            mdst = m_hbm.at[pl.ds(wid * CPC, CPC), pl.ds(blk * BLKN, BLKN)]

            @pl.when(blk >= 2)
            def _():
                pltpu.make_async_copy(mblk, mdst, sem_w).wait()

            sums = list(acc[:CPC])
            sqs = list(acc[CPC:])
            for g in range(GROUPS):
                goff = g * LANES

                def k_step(kq, kc):
                    mxs = list(kc[:2])
                    ss = list(kc[2:2 + CPC])
                    qq = list(kc[2 + CPC:])
                    sp = [bzero, bzero]
                    qp = [bzero, bzero]
                    for dk in range(4):
                        k = kq * 4 + dk
                        i1v = idxb[0, k, pl.ds(goff, LANES)]
                        i0v = idxb[1, k, pl.ds(goff, LANES)]
                        for p in range(2):
                            i1p = i1v if p == 0 else i1v + cnp
                            i0p = i0v if p == 0 else i0v + cnp
                            uw = plsc.load_gather(utile, [i1p])
                            vw = plsc.load_gather(vtile, [i0p])
                            s = (plsc.bitcast(uw, jnp.bfloat16)
                                 + plsc.bitcast(vw, jnp.bfloat16))
                            mxs[p] = jnp.maximum(mxs[p], s)
                            sp[p] = sp[p] + s
                            qp[p] = qp[p] + s * s
                    for p in range(2):
                        for (vec, accs) in ((sp[p], ss), (qp[p], qq)):
                            vi = plsc.bitcast(vec, jnp.int32)
                            lo = plsc.bitcast(vi << 16, jnp.float32)
                            hi = plsc.bitcast(vi & himask, jnp.float32)
                            accs[2 * p] = accs[2 * p] + lo
                            accs[2 * p + 1] = accs[2 * p + 1] + hi
                    return tuple(mxs + ss + qq)

                res = lax.fori_loop(0, K // 4, k_step,
                                    tuple([neg] * 2 + sums + sqs))
                sums = list(res[2:2 + CPC])
                sqs = list(res[2 + CPC:])
                for p in range(2):
                    mi = plsc.bitcast(res[p], jnp.int32)
                    mblk[2 * p, pl.ds(goff, LANES)] = plsc.bitcast(
                        mi << 16, jnp.float32)
                    mblk[2 * p + 1, pl.ds(goff, LANES)] = plsc.bitcast(
                        mi & himask, jnp.float32)
            pltpu.make_async_copy(mblk, mdst, sem_w).start()
            acc = tuple(sums + sqs)
        return acc

    fin = lax.fori_loop(0, NBLK // 2, blk_pair, tuple([zero] * (2 * CPC)))

    for c in range(CPC):
        statbuf[0, c, pl.ds(0, LANES)] = fin[c]
        statbuf[1, c, pl.ds(0, LANES)] = fin[CPC + c]
    pltpu.sync_copy(statbuf.at[0], ps_hbm.at[wid])
    pltpu.sync_copy(statbuf.at[1], pq_hbm.at[wid])

    for blk in (NBLK - 2, NBLK - 1):
        pltpu.make_async_copy(
            mbufs[blk % 2],
            m_hbm.at[pl.ds(wid * CPC, CPC), pl.ds(blk * BLKN, BLKN)],
            sem_w).wait()


def _bn_body(m_ref, ps_ref, pq_ref, g_ref, b_ref, o_ref):
    tot = jnp.float32(N * K)
    s = jnp.sum(ps_ref[...], axis=1, keepdims=True)
    q = jnp.sum(pq_ref[...], axis=1, keepdims=True)
    mean = s / tot
    var = q / tot - mean * mean
    a = lax.rsqrt(var + 1e-5) * g_ref[...]
    b = b_ref[...] - mean * a
    y = jnp.maximum(m_ref[...] * a + b, 0.0)
    o_ref[...] = y[:, :N].reshape(1, OUT, N)


@functools.cache
def _sc_call():
    return pl.kernel(
        _sc_body,
        out_type=(
            jax.ShapeDtypeStruct((OUT, NPAD), jnp.float32),
            jax.ShapeDtypeStruct((NT, CPC, LANES), jnp.float32),
            jax.ShapeDtypeStruct((NT, CPC, LANES), jnp.float32),
        ),
        mesh=plsc.VectorSubcoreMesh(core_axis_name="c",
                                    subcore_axis_name="s",
                                    num_cores=NC, num_subcores=NS),
        compiler_params=pltpu.CompilerParams(needs_layout_passes=False),
        scratch_types=[
            pltpu.VMEM((2 * NPAD,), jnp.int32),
            pltpu.VMEM((2 * NPAD,), jnp.int32),
            pltpu.VMEM((2, K, BLKN), jnp.int32),
            pltpu.VMEM((2, K, BLKN), jnp.int32),
            pltpu.VMEM((CPC, BLKN), jnp.float32),
            pltpu.VMEM((CPC, BLKN), jnp.float32),
            pltpu.VMEM((2, CPC, LANES), jnp.float32),
            pltpu.SemaphoreType.DMA,
            pltpu.SemaphoreType.DMA,
            pltpu.SemaphoreType.DMA,
        ],
    )


def kernel(x, edge_index, W, gamma, beta):
    xf = x.reshape(C, N)
    xp = jnp.pad(xf, ((0, 0), (0, NPAD - N)))

    i1 = edge_index[1].reshape(N, K)
    i0 = edge_index[0].reshape(N, K)
    pad = jnp.full((NPAD - N, K), N, dtype=jnp.int32)
    i1p = jnp.concatenate([i1, pad])
    i0p = jnp.concatenate([i0, pad])

    grid = NPAD // BN_BLK
    nsub = BN_BLK // BLKN
    ut, vt, idxb = pl.pallas_call(
        _tables_body,
        grid=(grid,),
        in_specs=[
            pl.BlockSpec((C, BN_BLK), lambda i: (0, i)),
            pl.BlockSpec((OUT // 2, 2 * C), lambda i: (0, 0)),
            pl.BlockSpec((OUT // 2, 2 * C), lambda i: (0, 0)),
            pl.BlockSpec((BN_BLK, K), lambda i: (i, 0)),
            pl.BlockSpec((BN_BLK, K), lambda i: (i, 0)),
        ],
        out_specs=[
            pl.BlockSpec((OUT // 2, BN_BLK), lambda i: (0, i)),
            pl.BlockSpec((OUT // 2, BN_BLK), lambda i: (0, i)),
            pl.BlockSpec((nsub, 2, K, BLKN), lambda i: (i, 0, 0, 0)),
        ],
        out_shape=[
            jax.ShapeDtypeStruct((OUT // 2, NPAD), jnp.int32),
            jax.ShapeDtypeStruct((OUT // 2, NPAD), jnp.int32),
            jax.ShapeDtypeStruct((NBLK, 2, K, BLKN), jnp.int32),
        ],
    )(xp, W[0::2], W[1::2], i1p, i0p)

    m, ps, pq = _sc_call()(idxb, ut, vt)

    outw = pl.pallas_call(
        _bn_body,
        in_specs=[
            pl.BlockSpec((OUT, NPAD), lambda: (0, 0)),
            pl.BlockSpec((OUT, LANES), lambda: (0, 0)),
            pl.BlockSpec((OUT, LANES), lambda: (0, 0)),
            pl.BlockSpec((OUT, 1), lambda: (0, 0)),
            pl.BlockSpec((OUT, 1), lambda: (0, 0)),
        ],
        out_specs=pl.BlockSpec((1, OUT, N), lambda: (0, 0, 0)),
        out_shape=jax.ShapeDtypeStruct((1, OUT, N), jnp.float32),
    )(m, ps.reshape(OUT, LANES), pq.reshape(OUT, LANES),
      gamma.reshape(OUT, 1), beta.reshape(OUT, 1))

    return outw


# speedup vs baseline: 1.0753x; 1.0753x over previous
import functools

import jax
import jax.numpy as jnp
from jax import lax
from jax.experimental import pallas as pl
from jax.experimental.pallas import tpu as pltpu
from jax.experimental.pallas import tpu_sc as plsc

NC, NS, LANES = 2, 16, 16
NT = NC * NS

N, C, K, OUT = 10000, 128, 32, 128
NPAD = 10240
CPC = OUT // NT
BLKN = 128
NBLK = NPAD // BLKN
GROUPS = BLKN // LANES
BN_BLK = 1024
CBLK = 8


def _tables_body(xp_ref, we_ref, wo_ref, ut_ref, vt_ref):
    we = we_ref[...]
    wo = wo_ref[...]
    xb = xp_ref[...]
    dn = (((1,), (0,)), ((), ()))

    def pack(mat_e, mat_o):
        e = lax.dot_general(mat_e, xb, dn, preferred_element_type=jnp.float32)
        o = lax.dot_general(mat_o, xb, dn, preferred_element_type=jnp.float32)
        el = lax.bitcast_convert_type(e.astype(jnp.bfloat16),
                                      jnp.uint16).astype(jnp.uint32)
        oh = lax.bitcast_convert_type(o.astype(jnp.bfloat16),
                                      jnp.uint16).astype(jnp.uint32)
        return (el | (oh << 16)).astype(jnp.int32)

    ut_ref[...] = pack(we[:, :C] - we[:, C:], wo[:, :C] - wo[:, C:])
    vt_ref[...] = pack(we[:, C:], wo[:, C:])


def _sc_body(idx_hbm, ut_hbm, vt_hbm, m_hbm, ps_hbm, pq_hbm,
             utile, vtile, idx0, idx1, mb0, mb1, statbuf,
             sem_i0, sem_i1, sem_w):
    cid = lax.axis_index("c")
    sid = lax.axis_index("s")
    wid = sid * NC + cid

    pltpu.sync_copy(ut_hbm.at[2 * wid], utile.at[pl.ds(0, NPAD)])
    pltpu.sync_copy(ut_hbm.at[2 * wid + 1], utile.at[pl.ds(NPAD, NPAD)])
    pltpu.sync_copy(vt_hbm.at[2 * wid], vtile.at[pl.ds(0, NPAD)])
    pltpu.sync_copy(vt_hbm.at[2 * wid + 1], vtile.at[pl.ds(NPAD, NPAD)])

    ibufs = ((idx0, sem_i0), (idx1, sem_i1))
    mbufs = (mb0, mb1)

    def idx_start(blk, slot):
        buf, sem = ibufs[slot]
        pltpu.make_async_copy(idx_hbm.at[blk], buf, sem).start()

    def idx_drain(blk, slot):
        buf, sem = ibufs[slot]
        pltpu.make_async_copy(idx_hbm.at[blk], buf, sem).wait()

    zero = jnp.zeros((LANES,), jnp.float32)
    neg = jnp.full((2 * LANES,), -jnp.inf, jnp.bfloat16)
    cnp = jnp.full((LANES,), NPAD, jnp.int32)
    himask = jnp.int32(-65536)
    bzero = jnp.zeros((2 * LANES,), jnp.bfloat16)

    idx_start(0, 0)

    def blk_pair(bg, carry):
        acc = carry
        for slot in range(2):
            blk = bg * 2 + slot
            bn = jnp.minimum(blk + 1, NBLK - 1)
            idx_start(bn, 1 - slot)
            idx_drain(blk, slot)
            idxb, _ = ibufs[slot]
            mblk = mbufs[slot]
            mdst = m_hbm.at[pl.ds(wid * CPC, CPC), pl.ds(blk * BLKN, BLKN)]

            @pl.when(blk >= 2)
            def _():
                pltpu.make_async_copy(mblk, mdst, sem_w).wait()

            sums = list(acc[:CPC])
            sqs = list(acc[CPC:])
            for g in range(GROUPS):
                goff = g * LANES

                def k_step(kq, kc):
                    mxs = list(kc[:2])
                    ss = list(kc[2:2 + CPC])
                    qq = list(kc[2 + CPC:])
                    sp = [bzero, bzero]
                    qp = [bzero, bzero]
                    for dk in range(4):
                        k = kq * 4 + dk
                        i1v = idxb[0, k, pl.ds(goff, LANES)]
                        i0v = idxb[1, k, pl.ds(goff, LANES)]
                        for p in range(2):
                            i1p = i1v if p == 0 else i1v + cnp
                            i0p = i0v if p == 0 else i0v + cnp
                            uw = plsc.load_gather(utile, [i1p])
                            vw = plsc.load_gather(vtile, [i0p])
                            s = (plsc.bitcast(uw, jnp.bfloat16)
                                 + plsc.bitcast(vw, jnp.bfloat16))
                            mxs[p] = jnp.maximum(mxs[p], s)
                            sp[p] = sp[p] + s
                            qp[p] = qp[p] + s * s
                    for p in range(2):
                        for (vec, accs) in ((sp[p], ss), (qp[p], qq)):
                            vi = plsc.bitcast(vec, jnp.int32)
                            lo = plsc.bitcast(vi << 16, jnp.float32)
                            hi = plsc.bitcast(vi & himask, jnp.float32)
                            accs[2 * p] = accs[2 * p] + lo
                            accs[2 * p + 1] = accs[2 * p + 1] + hi
                    return tuple(mxs + ss + qq)

                res = lax.fori_loop(0, K // 4, k_step,
                                    tuple([neg] * 2 + sums + sqs))
                sums = list(res[2:2 + CPC])
                sqs = list(res[2 + CPC:])
                for p in range(2):
                    mi = plsc.bitcast(res[p], jnp.int32)
                    mblk[2 * p, pl.ds(goff, LANES)] = plsc.bitcast(
                        mi << 16, jnp.float32)
                    mblk[2 * p + 1, pl.ds(goff, LANES)] = plsc.bitcast(
                        mi & himask, jnp.float32)
            pltpu.make_async_copy(mblk, mdst, sem_w).start()
            acc = tuple(sums + sqs)
        return acc

    fin = lax.fori_loop(0, NBLK // 2, blk_pair, tuple([zero] * (2 * CPC)))

    for c in range(CPC):
        statbuf[0, c, pl.ds(0, LANES)] = fin[c]
        statbuf[1, c, pl.ds(0, LANES)] = fin[CPC + c]
    pltpu.sync_copy(statbuf.at[0], ps_hbm.at[wid])
    pltpu.sync_copy(statbuf.at[1], pq_hbm.at[wid])

    for blk in (NBLK - 2, NBLK - 1):
        pltpu.make_async_copy(
            mbufs[blk % 2],
            m_hbm.at[pl.ds(wid * CPC, CPC), pl.ds(blk * BLKN, BLKN)],
            sem_w).wait()


def _bn_body(m_ref, ps_ref, pq_ref, g_ref, b_ref, o_ref):
    tot = jnp.float32(N * K)
    s = jnp.sum(ps_ref[...], axis=1, keepdims=True)
    q = jnp.sum(pq_ref[...], axis=1, keepdims=True)
    mean = s / tot
    var = q / tot - mean * mean
    a = lax.rsqrt(var + 1e-5) * g_ref[...]
    b = b_ref[...] - mean * a
    y = jnp.maximum(m_ref[...] * a + b, 0.0)
    o_ref[...] = y[:, :N].reshape(1, OUT, N)


@functools.cache
def _sc_call():
    return pl.kernel(
        _sc_body,
        out_type=(
            jax.ShapeDtypeStruct((OUT, NPAD), jnp.float32),
            jax.ShapeDtypeStruct((NT, CPC, LANES), jnp.float32),
            jax.ShapeDtypeStruct((NT, CPC, LANES), jnp.float32),
        ),
        mesh=plsc.VectorSubcoreMesh(core_axis_name="c",
                                    subcore_axis_name="s",
                                    num_cores=NC, num_subcores=NS),
        compiler_params=pltpu.CompilerParams(needs_layout_passes=False),
        scratch_types=[
            pltpu.VMEM((2 * NPAD,), jnp.int32),
            pltpu.VMEM((2 * NPAD,), jnp.int32),
            pltpu.VMEM((2, K, BLKN), jnp.int32),
            pltpu.VMEM((2, K, BLKN), jnp.int32),
            pltpu.VMEM((CPC, BLKN), jnp.float32),
            pltpu.VMEM((CPC, BLKN), jnp.float32),
            pltpu.VMEM((2, CPC, LANES), jnp.float32),
            pltpu.SemaphoreType.DMA,
            pltpu.SemaphoreType.DMA,
            pltpu.SemaphoreType.DMA,
        ],
    )


def kernel(x, edge_index, W, gamma, beta):
    xf = x.reshape(C, N)
    xp = jnp.pad(xf, ((0, 0), (0, NPAD - N)))

    i1 = edge_index[1].reshape(N, K)
    i0 = edge_index[0].reshape(N, K)
    pad = jnp.full((NPAD - N, K), N, dtype=jnp.int32)
    i1p = jnp.concatenate([i1, pad])
    i0p = jnp.concatenate([i0, pad])
    idxt = jnp.stack([i1p.T, i0p.T])
    idxb = idxt.reshape(2, K, NBLK, BLKN).transpose(2, 0, 1, 3)

    grid = NPAD // BN_BLK
    ut, vt = pl.pallas_call(
        _tables_body,
        grid=(grid,),
        in_specs=[
            pl.BlockSpec((C, BN_BLK), lambda i: (0, i)),
            pl.BlockSpec((OUT // 2, 2 * C), lambda i: (0, 0)),
            pl.BlockSpec((OUT // 2, 2 * C), lambda i: (0, 0)),
        ],
        out_specs=[
            pl.BlockSpec((OUT // 2, BN_BLK), lambda i: (0, i)),
            pl.BlockSpec((OUT // 2, BN_BLK), lambda i: (0, i)),
        ],
        out_shape=[
            jax.ShapeDtypeStruct((OUT // 2, NPAD), jnp.int32),
            jax.ShapeDtypeStruct((OUT // 2, NPAD), jnp.int32),
        ],
    )(xp, W[0::2], W[1::2])

    m, ps, pq = _sc_call()(idxb, ut, vt)

    outw = pl.pallas_call(
        _bn_body,
        in_specs=[
            pl.BlockSpec((OUT, NPAD), lambda: (0, 0)),
            pl.BlockSpec((OUT, LANES), lambda: (0, 0)),
            pl.BlockSpec((OUT, LANES), lambda: (0, 0)),
            pl.BlockSpec((OUT, 1), lambda: (0, 0)),
            pl.BlockSpec((OUT, 1), lambda: (0, 0)),
        ],
        out_specs=pl.BlockSpec((1, OUT, N), lambda: (0, 0, 0)),
        out_shape=jax.ShapeDtypeStruct((1, OUT, N), jnp.float32),
    )(m, ps.reshape(OUT, LANES), pq.reshape(OUT, LANES),
      gamma.reshape(OUT, 1), beta.reshape(OUT, 1))

    return outw
